# baseline (speedup 1.0000x reference)
import functools
import jax
import jax.numpy as jnp
from jax import lax
from jax.experimental import pallas as pl
from jax.experimental.pallas import tpu as pltpu
from jax.experimental.pallas import tpu_sc as plsc

_NBINS = 1024
_L = 16
_NC, _NS = 2, 16
_NW = _NC * _NS
_BLK = 32768
_UNROLL = 16
_G = 8192
_GPAD = _G + 16


def _ps_body(x_hbm, h_hbm, d_hbm, t_hbm, out_hbm,
             h_v, h0_v, f_v, d_v, t_v, cell_v, pk_v, xbuf, obuf):
    wid = lax.axis_index("s") * _NC + lax.axis_index("c")
    n = x_hbm.shape[0]
    per_w = n // _NW
    nblk = per_w // _BLK

    pltpu.sync_copy(h_hbm, h_v)
    pltpu.sync_copy(d_hbm, d_v)
    pltpu.sync_copy(t_hbm, t_v)

    lane = lax.iota(jnp.int32, _L)
    d_vec = d_v[...]
    t_vec = t_v[...]
    b_s = d_vec[0]

    def build_body(g, _):
        rows = (lane + g * _L) * 9
        h0 = plsc.load_gather(h_v, [rows])
        h0_v[pl.ds(g * _L, _L)] = h0
        acc = jnp.zeros((_L,), jnp.float32)
        for t in range(1, 9):
            if t == 1:
                v = h0
            else:
                v = plsc.load_gather(h_v, [rows + t])
            z = (v - t_vec[t] >= 0).astype(jnp.float32)
            acc = acc + z * d_vec[t]
        f_v[pl.ds(g * _L, _L)] = acc - b_s
        return 0

    lax.fori_loop(0, _NBINS // _L, build_body, 0)

    lo_s = d_vec[9]
    inv_s = d_vec[10]

    def cellof(v):
        t = jnp.clip((v - lo_s) * inv_s, 0.0, jnp.float32(_G - 1))
        return t.astype(jnp.int32)

    def cell_body(g, _):
        v = h0_v[pl.ds(g * _L, _L)]
        cell_v[pl.ds(g * _L, _L)] = cellof(v)
        return 0

    lax.fori_loop(0, _NBINS // _L, cell_body, 0)

    def base_body(g, _):
        cq = lane + g * _L
        cnt = jnp.zeros((_L,), jnp.int32)
        s = _NBINS // 2
        while s >= 1:
            pv = plsc.load_gather(cell_v, [cnt + (s - 1)])
            cnt = cnt + jnp.int32(s) * (pv < cq).astype(jnp.int32)
            s //= 2
        pk_v[pl.ds(g * _L, _L)] = cnt
        return 0

    lax.fori_loop(0, _GPAD // _L, base_body, 0)

    def pack_body(g, mx):
        b_cur = pk_v[pl.ds(g * _L, _L)]
        b_nxt = plsc.load_gather(pk_v, [lane + g * _L + 1])
        occ = b_nxt - b_cur
        pk_v[pl.ds(g * _L, _L)] = b_cur * 16 + jnp.minimum(occ, 15)
        return jnp.maximum(mx, occ)

    mx = lax.fori_loop(0, _G // _L, pack_body, jnp.zeros((_L,), jnp.int32))
    fast_ok = jnp.max(mx) <= 7

    def finish(xv, idx, off):
        left = plsc.load_gather(h0_v, [idx - 1])
        right = plsc.load_gather(h0_v, [idx])
        go_left = jnp.abs(xv - left) < jnp.abs(xv - right)
        nidx = idx - go_left.astype(jnp.int32)
        obuf[pl.ds(off, _L)] = plsc.load_gather(f_v, [nidx])

    def fast_vecs():
        @plsc.parallel_loop(0, _BLK // _L, unroll=_UNROLL)
        def vec_body(j):
            off = j * _L
            xv = xbuf[pl.ds(off, _L)]
            c = cellof(xv)
            pk = plsc.load_gather(pk_v, [c])
            b0 = lax.shift_right_logical(pk, 4)
            occ = pk & 15
            rel = jnp.zeros((_L,), jnp.int32)
            for s in (4, 2, 1):
                probe = jnp.minimum(b0 + rel + (s - 1), jnp.int32(_NBINS - 1))
                pv = plsc.load_gather(h0_v, [probe])
                take = ((rel + s) <= occ) & (pv < xv)
                rel = rel + jnp.int32(s) * take.astype(jnp.int32)
            idx = jnp.clip(b0 + rel, 1, _NBINS - 1)
            finish(xv, idx, off)

    def slow_vecs():
        @plsc.parallel_loop(0, _BLK // _L, unroll=_UNROLL)
        def vec_body(j):
            off = j * _L
            xv = xbuf[pl.ds(off, _L)]
            cnt = jnp.zeros((_L,), jnp.int32)
            s = _NBINS // 2
            while s >= 1:
                pv = plsc.load_gather(h0_v, [cnt + (s - 1)])
                cnt = cnt + jnp.int32(s) * (pv < xv).astype(jnp.int32)
                s //= 2
            idx = jnp.maximum(cnt, 1)
            finish(xv, idx, off)

    def run_blocks(vec_loop):
        def blk_body(bi, _):
            base_el = wid * per_w + bi * _BLK
            pltpu.sync_copy(x_hbm.at[pl.ds(base_el, _BLK)], xbuf)
            vec_loop()
            pltpu.sync_copy(obuf, out_hbm.at[pl.ds(base_el, _BLK)])
            return 0
        lax.fori_loop(0, nblk, blk_body, 0)

    lax.cond(fast_ok,
             lambda: run_blocks(fast_vecs),
             lambda: run_blocks(slow_vecs))


def _make_call(n, interpret=False):
    return pl.kernel(
        _ps_body,
        out_type=jax.ShapeDtypeStruct((n,), jnp.float32),
        mesh=plsc.VectorSubcoreMesh(
            core_axis_name="c", subcore_axis_name="s",
            num_cores=_NC, num_subcores=_NS),
        scratch_types=[
            pltpu.VMEM((_NBINS * 9,), jnp.float32),
            pltpu.VMEM((_NBINS,), jnp.float32),
            pltpu.VMEM((_NBINS,), jnp.float32),
            pltpu.VMEM((_L,), jnp.float32),
            pltpu.VMEM((_L,), jnp.float32),
            pltpu.VMEM((_NBINS,), jnp.int32),
            pltpu.VMEM((_GPAD,), jnp.int32),
            pltpu.VMEM((_BLK,), jnp.float32),
            pltpu.VMEM((_BLK,), jnp.float32),
        ],
        compiler_params=pltpu.CompilerParams(needs_layout_passes=False),
        interpret=interpret,
    )


@jax.jit
def _run(x, h, d, T, b):
    xf = x.reshape(-1)
    hf = h.reshape(-1)
    span = h[_NBINS - 1, 0] - h[0, 0]
    inv = jnp.where(span > 0, jnp.float32(_G) / span, jnp.float32(0.0))
    dq = (jnp.zeros((_L,), jnp.float32).at[:9].set(d).at[0].set(b)
          .at[9].set(h[0, 0]).at[10].set(inv))
    tq = jnp.zeros((_L,), jnp.float32).at[:9].set(T)
    out = _make_call(xf.shape[0])(xf, hf, dq, tq)
    return out.reshape(x.shape)


def kernel(x, h, d, T, b):
    return _run(x, h, d, T, b)


# speedup vs baseline: 1.6102x; 1.6102x over previous
import functools
import numpy as np
import jax
import jax.numpy as jnp
from jax import lax
from jax.experimental import pallas as pl
from jax.experimental.pallas import tpu as pltpu
from jax.experimental.pallas import tpu_sc as plsc

_NBINS = 1024
_L = 16
_NC, _NS = 2, 16
_NW = _NC * _NS
_BLK = 16384
_UNROLL = 8
_G = 8192
_GPAD = _G + 16

_TOP = np.uint32(0x80000000)


def _tokey(u):
    neg = u >= _TOP
    return jnp.where(neg, ~u, u | _TOP)


def _fromkey(k):
    neg = k < _TOP
    return jnp.where(neg, ~k, k ^ _TOP)


def _ps_body(x_hbm, h_hbm, d_hbm, t_hbm, out_hbm,
             h_v, h0_v, f_v, xs_v, d_v, t_v, cell_v, pk_v, xbuf, obuf):
    wid = lax.axis_index("s") * _NC + lax.axis_index("c")
    n = x_hbm.shape[0]
    per_w = n // _NW
    nblk = per_w // _BLK

    pltpu.sync_copy(h_hbm, h_v)
    pltpu.sync_copy(d_hbm, d_v)
    pltpu.sync_copy(t_hbm, t_v)

    lane = lax.iota(jnp.int32, _L)
    d_vec = d_v[...]
    t_vec = t_v[...]
    b_s = d_vec[0]
    lo_s = d_vec[9]
    inv_s = d_vec[10]

    def build_body(g, _):
        rows = (lane + g * _L) * 9
        h0 = plsc.load_gather(h_v, [rows])
        h0_v[pl.ds(g * _L, _L)] = h0
        acc = jnp.zeros((_L,), jnp.float32)
        for t in range(1, 9):
            if t == 1:
                v = h0
            else:
                v = plsc.load_gather(h_v, [rows + t])
            z = (v - t_vec[t] >= 0).astype(jnp.float32)
            acc = acc + z * d_vec[t]
        f_v[pl.ds(g * _L, _L)] = acc - b_s
        return 0

    lax.fori_loop(0, _NBINS // _L, build_body, 0)

    def xs_body(g, _):
        j = lane + g * _L
        l = plsc.load_gather(h0_v, [j])
        r = plsc.load_gather(h0_v, [jnp.minimum(j + 1, _NBINS - 1)])
        kl = _tokey(plsc.bitcast(l, jnp.uint32))
        kr = _tokey(plsc.bitcast(r, jnp.uint32))

        def bis(_, carry):
            klo, khi = carry
            active = (khi - klo) > 1
            mid = klo + lax.shift_right_logical(khi - klo, jnp.uint32(1))
            xm = plsc.bitcast(_fromkey(mid), jnp.float32)
            p_right = ~(jnp.abs(xm - l) < jnp.abs(xm - r))
            khi = jnp.where(active & p_right, mid, khi)
            klo = jnp.where(active & ~p_right, mid, klo)
            return klo, khi

        _, khi = lax.fori_loop(0, 32, bis, (kl, kr))
        ks = jnp.where(kl == kr, kr + 1, khi)
        xs = plsc.bitcast(_fromkey(ks), jnp.float32)
        inf = plsc.bitcast(jnp.full((_L,), 0x7F800000, jnp.uint32), jnp.float32)
        xs_v[pl.ds(g * _L, _L)] = jnp.where(j > _NBINS - 2, inf, xs)
        return 0

    lax.fori_loop(0, _NBINS // _L, xs_body, 0)

    def cellof(v):
        t = jnp.clip((v - lo_s) * inv_s, 0.0, jnp.float32(_G - 1))
        return t.astype(jnp.int32)

    def cell_body(g, _):
        v = xs_v[pl.ds(g * _L, _L)]
        cell_v[pl.ds(g * _L, _L)] = cellof(v)
        return 0

    lax.fori_loop(0, _NBINS // _L, cell_body, 0)

    def base_body(g, _):
        cq = lane + g * _L
        cnt = jnp.zeros((_L,), jnp.int32)
        s = _NBINS // 2
        while s >= 1:
            pv = plsc.load_gather(cell_v, [cnt + (s - 1)])
            cnt = cnt + jnp.int32(s) * (pv < cq).astype(jnp.int32)
            s //= 2
        pk_v[pl.ds(g * _L, _L)] = cnt
        return 0

    lax.fori_loop(0, _GPAD // _L, base_body, 0)

    def pack_body(g, mx):
        b_cur = pk_v[pl.ds(g * _L, _L)]
        b_nxt = plsc.load_gather(pk_v, [lane + g * _L + 1])
        occ = b_nxt - b_cur
        pk_v[pl.ds(g * _L, _L)] = b_cur * 16 + jnp.minimum(occ, 15)
        return jnp.maximum(mx, occ)

    mx = lax.fori_loop(0, _G // _L, pack_body, jnp.zeros((_L,), jnp.int32))
    fast_ok = jnp.max(mx) <= 7

    def fast_vecs():
        @plsc.parallel_loop(0, _BLK // _L, unroll=_UNROLL)
        def vec_body(j):
            off = j * _L
            xv = xbuf[pl.ds(off, _L)]
            c = cellof(xv)
            pk = plsc.load_gather(pk_v, [c])
            b0 = lax.shift_right_logical(pk, 4)
            occ = pk & 15
            rel = jnp.zeros((_L,), jnp.int32)
            for s in (4, 2, 1):
                probe = jnp.minimum(b0 + rel + (s - 1), jnp.int32(_NBINS - 1))
                pv = plsc.load_gather(xs_v, [probe])
                take = ((rel + s) <= occ) & (pv <= xv)
                rel = rel + jnp.int32(s) * take.astype(jnp.int32)
            obuf[pl.ds(off, _L)] = plsc.load_gather(f_v, [b0 + rel])

    def slow_vecs():
        @plsc.parallel_loop(0, _BLK // _L, unroll=_UNROLL)
        def vec_body(j):
            off = j * _L
            xv = xbuf[pl.ds(off, _L)]
            cnt = jnp.zeros((_L,), jnp.int32)
            s = _NBINS // 2
            while s >= 1:
                pv = plsc.load_gather(h0_v, [cnt + (s - 1)])
                cnt = cnt + jnp.int32(s) * (pv < xv).astype(jnp.int32)
                s //= 2
            idx = jnp.maximum(cnt, 1)
            left = plsc.load_gather(h0_v, [idx - 1])
            right = plsc.load_gather(h0_v, [idx])
            go_left = jnp.abs(xv - left) < jnp.abs(xv - right)
            nidx = idx - go_left.astype(jnp.int32)
            obuf[pl.ds(off, _L)] = plsc.load_gather(f_v, [nidx])

    def run_blocks(vec_loop):
        def blk_body(bi, _):
            base_el = wid * per_w + bi * _BLK
            pltpu.sync_copy(x_hbm.at[pl.ds(base_el, _BLK)], xbuf)
            vec_loop()
            pltpu.sync_copy(obuf, out_hbm.at[pl.ds(base_el, _BLK)])
            return 0
        lax.fori_loop(0, nblk, blk_body, 0)

    lax.cond(fast_ok,
             lambda: run_blocks(fast_vecs),
             lambda: run_blocks(slow_vecs))


def _make_call(n, interpret=False):
    return pl.kernel(
        _ps_body,
        out_type=jax.ShapeDtypeStruct((n,), jnp.float32),
        mesh=plsc.VectorSubcoreMesh(
            core_axis_name="c", subcore_axis_name="s",
            num_cores=_NC, num_subcores=_NS),
        scratch_types=[
            pltpu.VMEM((_NBINS * 9,), jnp.float32),
            pltpu.VMEM((_NBINS,), jnp.float32),
            pltpu.VMEM((_NBINS,), jnp.float32),
            pltpu.VMEM((_NBINS,), jnp.float32),
            pltpu.VMEM((_L,), jnp.float32),
            pltpu.VMEM((_L,), jnp.float32),
            pltpu.VMEM((_NBINS,), jnp.int32),
            pltpu.VMEM((_GPAD,), jnp.int32),
            pltpu.VMEM((_BLK,), jnp.float32),
            pltpu.VMEM((_BLK,), jnp.float32),
        ],
        compiler_params=pltpu.CompilerParams(needs_layout_passes=False),
        interpret=interpret,
    )


@jax.jit
def _run(x, h, d, T, b):
    xf = x.reshape(-1)
    hf = h.reshape(-1)
    span = h[_NBINS - 1, 0] - h[0, 0]
    inv = jnp.where(span > 0, jnp.float32(_G) / span, jnp.float32(0.0))
    dq = (jnp.zeros((_L,), jnp.float32).at[:9].set(d).at[0].set(b)
          .at[9].set(h[0, 0]).at[10].set(inv))
    tq = jnp.zeros((_L,), jnp.float32).at[:9].set(T)
    out = _make_call(xf.shape[0])(xf, hf, dq, tq)
    return out.reshape(x.shape)


def kernel(x, h, d, T, b):
    return _run(x, h, d, T, b)
